# baseline (speedup 1.0000x reference)
import functools

import jax
import jax.numpy as jnp
from jax import lax
from jax.experimental import pallas as pl
from jax.experimental.pallas import tpu as pltpu, tpu_sc as plsc

N = 10000
NP = 10240
E = 320000
D = 128
G = 64
KSEL = N // 2
NCORES = 2
NSUB = 16
NTILES = NCORES * NSUB
EPT = E // NTILES
ECH = 128
NCH = (EPT + ECH - 1) // ECH
EPAD = NCH * ECH
DH = D // 2
EPT2 = E // NSUB
NCH2 = 158
EPAD2 = NCH2 * ECH
RPT = NP // NTILES
INT_MIN = -(2 ** 31)

_mesh = plsc.VectorSubcoreMesh(core_axis_name="c", subcore_axis_name="s")


@functools.partial(
    pl.kernel,
    out_type=jax.ShapeDtypeStruct((NTILES, NP), jnp.float32),
    mesh=_mesh,
    scratch_types=[
        pltpu.VMEM((EPAD,), jnp.int32),
        pltpu.VMEM((NP,), jnp.float32),
    ],
    compiler_params=pltpu.CompilerParams(needs_layout_passes=False),
)
def _sc_deg(dst_hbm, out_hbm, dstv, deg_l):
    c = lax.axis_index("c")
    s = lax.axis_index("s")
    wid = c * NSUB + s
    pltpu.sync_copy(dst_hbm.at[wid], dstv)
    def _zero(i, _):
        deg_l[pl.ds(i * 16, 16)] = jnp.zeros((16,), jnp.float32)
        return 0
    lax.fori_loop(0, NP // 16, _zero, 0)
    ones = jnp.ones((16,), jnp.float32)
    def _grp(g, _):
        ii = dstv[pl.ds(g * 16, 16)]
        plsc.addupdate_scatter(deg_l, [ii], ones)
        return 0
    lax.fori_loop(0, EPAD // 16, _grp, 0)
    pltpu.sync_copy(deg_l, out_hbm.at[wid])


@functools.partial(
    pl.kernel,
    out_type=jax.ShapeDtypeStruct((NCORES, NP, DH), jnp.float32),
    mesh=_mesh,
    scratch_types=[
        pltpu.VMEM((NCH2, ECH), jnp.int32),
        pltpu.VMEM((NCH2, ECH), jnp.int32),
        pltpu.VMEM((ECH, DH), jnp.float32),
        pltpu.VMEM((ECH, DH), jnp.float32),
        pltpu.VMEM_SHARED((NP, DH), jnp.float32),
        pltpu.SemaphoreType.DMA,
        pltpu.SemaphoreType.DMA,
    ],
    compiler_params=pltpu.CompilerParams(use_tc_tiling_on_sc=False),
)
def _sc_agg(z_hbm, src_hbm, dst_hbm, zero_hbm, out_hbm,
            srcv, dstv, rows0, rows1, agg_sh, sem0, sem1):
    c = lax.axis_index("c")
    s = lax.axis_index("s")
    zc = z_hbm.at[c]
    pltpu.sync_copy(src_hbm.at[s], srcv)
    pltpu.sync_copy(dst_hbm.at[s], dstv)
    pltpu.sync_copy(zero_hbm, agg_sh.at[pl.ds(s * (NP // NSUB), NP // NSUB)])
    plsc.subcore_barrier()
    pltpu.async_copy(zc.at[srcv.at[0]], rows0, sem0)
    def _pair(k, _):
        i0 = 2 * k
        pltpu.async_copy(zc.at[srcv.at[i0 + 1]], rows1, sem1)
        pltpu.make_async_copy(zc.at[srcv.at[i0]], rows0, sem0).wait()
        pltpu.sync_copy(rows0, agg_sh.at[dstv.at[i0]], add=True)
        pltpu.async_copy(zc.at[srcv.at[i0 + 2]], rows0, sem0)
        pltpu.make_async_copy(zc.at[srcv.at[i0 + 1]], rows1, sem1).wait()
        pltpu.sync_copy(rows1, agg_sh.at[dstv.at[i0 + 1]], add=True)
        return 0
    lax.fori_loop(0, (NCH2 - 2) // 2, _pair, 0)
    pltpu.make_async_copy(zc.at[srcv.at[NCH2 - 2]], rows0, sem0).wait()
    pltpu.sync_copy(rows0, agg_sh.at[dstv.at[NCH2 - 2]], add=True)
    pltpu.async_copy(zc.at[srcv.at[NCH2 - 1]], rows1, sem1)
    pltpu.make_async_copy(zc.at[srcv.at[NCH2 - 1]], rows1, sem1).wait()
    pltpu.sync_copy(rows1, agg_sh.at[dstv.at[NCH2 - 1]], add=True)
    plsc.subcore_barrier()
    pltpu.sync_copy(
        agg_sh.at[pl.ds(s * (NP // NSUB), NP // NSUB)],
        out_hbm.at[c, pl.ds(s * (NP // NSUB), NP // NSUB)],
    )


@functools.partial(
    pl.kernel,
    out_type=jax.ShapeDtypeStruct((NTILES, G * D), jnp.float32),
    mesh=_mesh,
    scratch_types=[
        pltpu.VMEM((RPT * D,), jnp.float32),
        pltpu.VMEM((RPT,), jnp.int32),
        pltpu.VMEM((G * D,), jnp.float32),
    ],
    compiler_params=pltpu.CompilerParams(needs_layout_passes=False),
)
def _sc_max(xw_hbm, batch_hbm, out_hbm, rowsf, batchv, acc_m):
    c = lax.axis_index("c")
    s = lax.axis_index("s")
    wid = c * NSUB + s
    pltpu.sync_copy(xw_hbm.at[pl.ds(wid * RPT * D, RPT * D)], rowsf)
    pltpu.sync_copy(batch_hbm.at[wid], batchv)
    def _zero(i, _):
        acc_m[pl.ds(i * 16, 16)] = jnp.zeros((16,), jnp.float32)
        return 0
    lax.fori_loop(0, G * D // 16, _zero, 0)
    def _group(g, _):
        bb = batchv[pl.ds(g * 16, 16)]
        for r in range(16):
            base = bb[r] * D
            i = g * 16 + r
            for j in range(D // 16):
                ch = rowsf[pl.ds(i * D + j * 16, 16)]
                idx = base + j * 16 + lax.iota(jnp.int32, 16)
                cur = plsc.load_gather(acc_m, [idx])
                plsc.store_scatter(acc_m, [idx], jnp.maximum(cur, ch))
        return 0
    lax.fori_loop(0, RPT // 16, _group, 0)
    pltpu.sync_copy(acc_m, out_hbm.at[wid])


def _tc_pre(dt_ref, x_ref, rdeg_ref, z_ref):
    deg = jnp.sum(dt_ref[...], axis=1, keepdims=True)
    rdeg = lax.rsqrt(jnp.maximum(deg, 1.0))
    rdeg_ref[...] = rdeg
    z = x_ref[...] * rdeg
    z_ref[0] = z[:, :DH]
    z_ref[1] = z[:, DH:]


def _tc_conv(sp_ref, rdeg_ref, w_ref, b_ref, o_ref, *, scale_out):
    a = jnp.concatenate([sp_ref[0], sp_ref[1]], axis=1) * rdeg_ref[...]
    h = jnp.dot(a, w_ref[...], preferred_element_type=jnp.float32)
    f = jnp.maximum(h + b_ref[...], 0.0)
    if scale_out:
        f = f * rdeg_ref[...]
        o_ref[0] = f[:, :DH]
        o_ref[1] = f[:, DH:]
    else:
        o_ref[...] = f


def _tc_thresh(f_ref, p_ref, batch_ref, xw_ref, sums_ref, cnts_ref):
    p = p_ref[...]
    pn = p * lax.rsqrt(jnp.sum(p * p))
    f = f_ref[...]
    score = jnp.sum(f * pn, axis=1, keepdims=True)
    u = lax.bitcast_convert_type(score, jnp.int32)
    key = jnp.where(u >= 0, u, u ^ jnp.int32(0x7FFFFFFF))
    ridx = lax.broadcasted_iota(jnp.int32, (NP, 1), 0)
    valid = ridx < N
    key = jnp.where(valid, key, jnp.int32(INT_MIN))

    def _bis(_, carry):
        lo, hi = carry
        mid = (lo & hi) + ((lo ^ hi) >> 1)
        cnt = jnp.sum((key >= mid).astype(jnp.int32))
        ge = cnt >= KSEL
        return (jnp.where(ge, mid, lo), jnp.where(ge, hi, mid))

    lo, _ = lax.fori_loop(0, 32, _bis, (jnp.int32(INT_MIN),
                                        jnp.int32(0x7F800000)))
    sel = jnp.logical_and(valid, key >= lo)
    w = jnp.where(sel, jnp.maximum(score, 0.0), 0.0)
    xw = f * w
    xw_ref[...] = xw
    gI = lax.broadcasted_iota(jnp.int32, (G, NP), 0)
    oh = (gI == batch_ref[...]).astype(jnp.float32)
    sums_ref[...] = jnp.dot(oh, xw, preferred_element_type=jnp.float32)
    cnts_ref[...] = jnp.dot(oh, sel.astype(jnp.float32),
                            preferred_element_type=jnp.float32)


def _tc_head(sums_ref, cnts_ref, maxp_ref, w1_ref, b1_ref, w2_ref, b2_ref,
             o_ref):
    m = maxp_ref[0]
    for i in range(1, NTILES):
        m = jnp.maximum(m, maxp_ref[i])
    m = jnp.maximum(m, 0.0)
    cnt = jnp.maximum(cnts_ref[...], 1.0)
    mean = jnp.maximum(sums_ref[...] / cnt, 0.0)
    readout = jnp.concatenate([mean, m], axis=1)
    h = jnp.dot(readout, w1_ref[...], preferred_element_type=jnp.float32)
    h = jnp.maximum(h + b1_ref[...], 0.0)
    o_ref[...] = jnp.dot(h, w2_ref[...],
                         preferred_element_type=jnp.float32) + b2_ref[...]


def _call_pre(degt, x_pad):
    return pl.pallas_call(
        _tc_pre,
        out_shape=(jax.ShapeDtypeStruct((NP, 1), jnp.float32),
                   jax.ShapeDtypeStruct((NCORES, NP, DH), jnp.float32)),
    )(degt, x_pad)


def _call_conv(sp, rdeg, w, b, scale_out):
    out_shape = (jax.ShapeDtypeStruct((NCORES, NP, DH), jnp.float32)
                 if scale_out else jax.ShapeDtypeStruct((NP, D), jnp.float32))
    return pl.pallas_call(
        functools.partial(_tc_conv, scale_out=scale_out),
        out_shape=out_shape,
    )(sp, rdeg, w, b.reshape(1, D))


def _call_thresh(f5, p, batch_row):
    return pl.pallas_call(
        _tc_thresh,
        out_shape=(jax.ShapeDtypeStruct((NP, D), jnp.float32),
                   jax.ShapeDtypeStruct((G, D), jnp.float32),
                   jax.ShapeDtypeStruct((G, 1), jnp.float32)),
    )(f5, p.reshape(1, D), batch_row)


def _call_head(sums, cnts, maxp, w1, b1, w2, b2):
    return pl.pallas_call(
        _tc_head,
        out_shape=jax.ShapeDtypeStruct((G, 10), jnp.float32),
    )(sums, cnts, maxp, w1, b1.reshape(1, D), w2, b2.reshape(1, 10))


def kernel(x, edge_index, batch, Wt0, bt0, Wt1, bt1, Wg00, bg00, Wg01, bg01,
           Wg10, bg10, Wg11, bg11, p_score, Wc1, bc1, Wc2, bc2):
    src = edge_index[0].reshape(NSUB, EPT2)
    dst = edge_index[1].reshape(NSUB, EPT2)
    src_slab = jnp.pad(src, ((0, 0), (0, EPAD2 - EPT2))).reshape(
        NSUB, NCH2, ECH)
    dst_slab = jnp.pad(dst, ((0, 0), (0, EPAD2 - EPT2)),
                       constant_values=N).reshape(NSUB, NCH2, ECH)
    x_pad = jnp.pad(x, ((0, NP - N), (0, 0)))
    zero_d = jnp.zeros((NP // NSUB, DH), jnp.float32)

    dst_deg = jnp.pad(edge_index[1].reshape(NTILES, EPT),
                      ((0, 0), (0, EPAD - EPT)), constant_values=N)
    degp = _sc_deg(dst_deg)
    rdeg, z = _call_pre(degp.T, x_pad)

    convs = ((Wt0, bt0, True), (Wt1, bt1, True),
             (Wg00, bg00, True), (Wg01, bg01, False))
    for w, b, scale_out in convs:
        sp = _sc_agg(z, src_slab, dst_slab, zero_d)
        z = _call_conv(sp, rdeg, w, b, scale_out)

    batch_pad = jnp.pad(batch, (0, NP - N), constant_values=G - 1)
    xw, sums, cnts = _call_thresh(z, p_score, batch_pad.reshape(1, NP))
    maxp = _sc_max(xw.reshape(NP * D), batch_pad.reshape(NTILES, RPT))
    out = _call_head(sums, cnts, maxp.reshape(NTILES, G, D),
                     Wc1, bc1, Wc2, bc2)
    return out


# speedup vs baseline: 1.3468x; 1.3468x over previous
import functools

import jax
import jax.numpy as jnp
from jax import lax
from jax.experimental import pallas as pl
from jax.experimental.pallas import tpu as pltpu, tpu_sc as plsc

N = 10000
NP = 10240
E = 320000
D = 128
G = 64
KSEL = N // 2
NCORES = 2
NSUB = 16
NTILES = NCORES * NSUB
EPT = E // NTILES
ECH = 128
NCH = (EPT + ECH - 1) // ECH
EPAD = NCH * ECH
DH = D // 2
EPT2 = E // NSUB
NCH2 = 158
EPAD2 = NCH2 * ECH
RPT = NP // NTILES
INT_MIN = -(2 ** 31)

_mesh = plsc.VectorSubcoreMesh(core_axis_name="c", subcore_axis_name="s")


@functools.partial(
    pl.kernel,
    out_type=jax.ShapeDtypeStruct((NTILES, NP), jnp.float32),
    mesh=_mesh,
    scratch_types=[
        pltpu.VMEM((EPAD,), jnp.int32),
        pltpu.VMEM((NP,), jnp.float32),
    ],
    compiler_params=pltpu.CompilerParams(needs_layout_passes=False),
)
def _sc_deg(dst_hbm, out_hbm, dstv, deg_l):
    c = lax.axis_index("c")
    s = lax.axis_index("s")
    wid = c * NSUB + s
    pltpu.sync_copy(dst_hbm.at[wid], dstv)
    def _zero(i, _):
        deg_l[pl.ds(i * 16, 16)] = jnp.zeros((16,), jnp.float32)
        return 0
    lax.fori_loop(0, NP // 16, _zero, 0)
    ones = jnp.ones((16,), jnp.float32)
    def _grp(g, _):
        ii = dstv[pl.ds(g * 16, 16)]
        plsc.addupdate_scatter(deg_l, [ii], ones)
        return 0
    lax.fori_loop(0, EPAD // 16, _grp, 0)
    pltpu.sync_copy(deg_l, out_hbm.at[wid])


@functools.partial(
    pl.kernel,
    out_type=jax.ShapeDtypeStruct((NCORES, NP, DH), jnp.float32),
    mesh=_mesh,
    scratch_types=[
        pltpu.VMEM((NCH2, ECH), jnp.int32),
        pltpu.VMEM((NCH2, ECH), jnp.int32),
        pltpu.VMEM((ECH, DH), jnp.float32),
        pltpu.VMEM((ECH, DH), jnp.float32),
        pltpu.VMEM_SHARED((NP, DH), jnp.float32),
        pltpu.SemaphoreType.DMA,
        pltpu.SemaphoreType.DMA,
    ],
    compiler_params=pltpu.CompilerParams(use_tc_tiling_on_sc=False),
)
def _sc_agg(z_hbm, src_hbm, dst_hbm, zero_hbm, out_hbm,
            srcv, dstv, rows0, rows1, agg_sh, sem0, sem1):
    c = lax.axis_index("c")
    s = lax.axis_index("s")
    zc = z_hbm.at[c]
    pltpu.sync_copy(src_hbm.at[s], srcv)
    pltpu.sync_copy(dst_hbm.at[s], dstv)
    pltpu.sync_copy(zero_hbm, agg_sh.at[pl.ds(s * (NP // NSUB), NP // NSUB)])
    plsc.subcore_barrier()
    pltpu.async_copy(zc.at[srcv.at[0]], rows0, sem0)
    def _pair(k, _):
        i0 = 2 * k
        pltpu.async_copy(zc.at[srcv.at[i0 + 1]], rows1, sem1)
        pltpu.make_async_copy(zc.at[srcv.at[i0]], rows0, sem0).wait()
        pltpu.sync_copy(rows0, agg_sh.at[dstv.at[i0]], add=True)
        pltpu.async_copy(zc.at[srcv.at[i0 + 2]], rows0, sem0)
        pltpu.make_async_copy(zc.at[srcv.at[i0 + 1]], rows1, sem1).wait()
        pltpu.sync_copy(rows1, agg_sh.at[dstv.at[i0 + 1]], add=True)
        return 0
    lax.fori_loop(0, (NCH2 - 2) // 2, _pair, 0)
    pltpu.make_async_copy(zc.at[srcv.at[NCH2 - 2]], rows0, sem0).wait()
    pltpu.sync_copy(rows0, agg_sh.at[dstv.at[NCH2 - 2]], add=True)
    pltpu.async_copy(zc.at[srcv.at[NCH2 - 1]], rows1, sem1)
    pltpu.make_async_copy(zc.at[srcv.at[NCH2 - 1]], rows1, sem1).wait()
    pltpu.sync_copy(rows1, agg_sh.at[dstv.at[NCH2 - 1]], add=True)
    plsc.subcore_barrier()
    pltpu.sync_copy(
        agg_sh.at[pl.ds(s * (NP // NSUB), NP // NSUB)],
        out_hbm.at[c, pl.ds(s * (NP // NSUB), NP // NSUB)],
    )


@functools.partial(
    pl.kernel,
    out_type=jax.ShapeDtypeStruct((NTILES, G * D), jnp.float32),
    mesh=_mesh,
    scratch_types=[
        pltpu.VMEM((RPT * D,), jnp.float32),
        pltpu.VMEM((RPT,), jnp.int32),
        pltpu.VMEM((G * D,), jnp.float32),
    ],
    compiler_params=pltpu.CompilerParams(needs_layout_passes=False),
)
def _sc_max(xw_hbm, batch_hbm, out_hbm, rowsf, batchv, acc_m):
    c = lax.axis_index("c")
    s = lax.axis_index("s")
    wid = c * NSUB + s
    pltpu.sync_copy(xw_hbm.at[pl.ds(wid * RPT * D, RPT * D)], rowsf)
    pltpu.sync_copy(batch_hbm.at[wid], batchv)
    def _zero(i, _):
        acc_m[pl.ds(i * 16, 16)] = jnp.zeros((16,), jnp.float32)
        return 0
    lax.fori_loop(0, G * D // 16, _zero, 0)
    def _group(g, _):
        bb = batchv[pl.ds(g * 16, 16)]
        for r in range(16):
            base = bb[r] * D
            i = g * 16 + r
            for j in range(D // 16):
                ch = rowsf[pl.ds(i * D + j * 16, 16)]
                idx = base + j * 16 + lax.iota(jnp.int32, 16)
                cur = plsc.load_gather(acc_m, [idx])
                plsc.store_scatter(acc_m, [idx], jnp.maximum(cur, ch))
        return 0
    lax.fori_loop(0, RPT // 16, _group, 0)
    pltpu.sync_copy(acc_m, out_hbm.at[wid])


def _tc_pre(dt_ref, x_ref, rdeg_ref, z_ref):
    deg = jnp.sum(dt_ref[...], axis=1, keepdims=True)
    rdeg = lax.rsqrt(jnp.maximum(deg, 1.0))
    rdeg_ref[...] = rdeg
    z = x_ref[...] * rdeg
    z_ref[0] = z[:, :DH]
    z_ref[1] = z[:, DH:]


def _tc_conv(sp_ref, rdeg_ref, w_ref, b_ref, o_ref, *, scale_out):
    a = jnp.concatenate([sp_ref[0], sp_ref[1]], axis=1) * rdeg_ref[...]
    h = jnp.dot(a, w_ref[...], preferred_element_type=jnp.float32)
    f = jnp.maximum(h + b_ref[...], 0.0)
    if scale_out:
        f = f * rdeg_ref[...]
        o_ref[0] = f[:, :DH]
        o_ref[1] = f[:, DH:]
    else:
        o_ref[...] = f


def _tc_thresh(f_ref, p_ref, batch_ref, xw_ref, sums_ref, cnts_ref):
    p = p_ref[...]
    pn = p * lax.rsqrt(jnp.sum(p * p))
    f = f_ref[...]
    score = jnp.sum(f * pn, axis=1, keepdims=True)
    u = lax.bitcast_convert_type(score, jnp.int32)
    key = jnp.where(u >= 0, u, u ^ jnp.int32(0x7FFFFFFF))
    ridx = lax.broadcasted_iota(jnp.int32, (NP, 1), 0)
    valid = ridx < N
    key = jnp.where(valid, key, jnp.int32(INT_MIN))

    def _bis(_, carry):
        lo, hi = carry
        mid = (lo & hi) + ((lo ^ hi) >> 1)
        cnt = jnp.sum((key >= mid).astype(jnp.int32))
        ge = cnt >= KSEL
        return (jnp.where(ge, mid, lo), jnp.where(ge, hi, mid))

    lo, _ = lax.fori_loop(0, 32, _bis, (jnp.int32(INT_MIN),
                                        jnp.int32(0x7F800000)))
    sel = jnp.logical_and(valid, key >= lo)
    w = jnp.where(sel, jnp.maximum(score, 0.0), 0.0)
    xw = f * w
    xw_ref[...] = xw
    gI = lax.broadcasted_iota(jnp.int32, (G, NP), 0)
    oh = (gI == batch_ref[...]).astype(jnp.float32)
    sums_ref[...] = jnp.dot(oh, xw, preferred_element_type=jnp.float32)
    cnts_ref[...] = jnp.dot(oh, sel.astype(jnp.float32),
                            preferred_element_type=jnp.float32)


def _tc_head(sums_ref, cnts_ref, maxp_ref, w1_ref, b1_ref, w2_ref, b2_ref,
             o_ref):
    m = maxp_ref[0]
    for i in range(1, NTILES):
        m = jnp.maximum(m, maxp_ref[i])
    m = jnp.maximum(m, 0.0)
    cnt = jnp.maximum(cnts_ref[...], 1.0)
    mean = jnp.maximum(sums_ref[...] / cnt, 0.0)
    readout = jnp.concatenate([mean, m], axis=1)
    h = jnp.dot(readout, w1_ref[...], preferred_element_type=jnp.float32)
    h = jnp.maximum(h + b1_ref[...], 0.0)
    o_ref[...] = jnp.dot(h, w2_ref[...],
                         preferred_element_type=jnp.float32) + b2_ref[...]


def _call_pre(degt, x_pad):
    return pl.pallas_call(
        _tc_pre,
        out_shape=(jax.ShapeDtypeStruct((NP, 1), jnp.float32),
                   jax.ShapeDtypeStruct((NCORES, NP, DH), jnp.float32)),
    )(degt, x_pad)


def _call_conv(sp, rdeg, w, b, scale_out):
    out_shape = (jax.ShapeDtypeStruct((NCORES, NP, DH), jnp.float32)
                 if scale_out else jax.ShapeDtypeStruct((NP, D), jnp.float32))
    return pl.pallas_call(
        functools.partial(_tc_conv, scale_out=scale_out),
        out_shape=out_shape,
    )(sp, rdeg, w, b.reshape(1, D))


def _call_thresh(f5, p, batch_row):
    return pl.pallas_call(
        _tc_thresh,
        out_shape=(jax.ShapeDtypeStruct((NP, D), jnp.float32),
                   jax.ShapeDtypeStruct((G, D), jnp.float32),
                   jax.ShapeDtypeStruct((G, 1), jnp.float32)),
    )(f5, p.reshape(1, D), batch_row)


def _call_head(sums, cnts, maxp, w1, b1, w2, b2):
    return pl.pallas_call(
        _tc_head,
        out_shape=jax.ShapeDtypeStruct((G, 10), jnp.float32),
    )(sums, cnts, maxp, w1, b1.reshape(1, D), w2, b2.reshape(1, 10))


def kernel(x, edge_index, batch, Wt0, bt0, Wt1, bt1, Wg00, bg00, Wg01, bg01,
           Wg10, bg10, Wg11, bg11, p_score, Wc1, bc1, Wc2, bc2):
    src = edge_index[0].reshape(NSUB, EPT2)
    dst = edge_index[1].reshape(NSUB, EPT2)
    pad_w = EPAD2 - EPT2
    pad_dst = (N + (jnp.arange(NSUB * pad_w, dtype=jnp.int32) % (NP - N))
               ).reshape(NSUB, pad_w)
    pad_src = (jnp.arange(NSUB * pad_w, dtype=jnp.int32) % N
               ).reshape(NSUB, pad_w)
    src_slab = jnp.concatenate([src, pad_src], axis=1).reshape(
        NSUB, NCH2, ECH)
    dst_slab = jnp.concatenate([dst, pad_dst], axis=1).reshape(
        NSUB, NCH2, ECH)
    x_pad = jnp.pad(x, ((0, NP - N), (0, 0)))
    zero_d = jnp.zeros((NP // NSUB, DH), jnp.float32)

    dst_deg = jnp.pad(edge_index[1].reshape(NTILES, EPT),
                      ((0, 0), (0, EPAD - EPT)), constant_values=N)
    degp = _sc_deg(dst_deg)
    rdeg, z = _call_pre(degp.T, x_pad)

    convs = ((Wt0, bt0, True), (Wt1, bt1, True),
             (Wg00, bg00, True), (Wg01, bg01, False))
    for w, b, scale_out in convs:
        sp = _sc_agg(z, src_slab, dst_slab, zero_d)
        z = _call_conv(sp, rdeg, w, b, scale_out)

    batch_pad = jnp.pad(batch, (0, NP - N), constant_values=G - 1)
    xw, sums, cnts = _call_thresh(z, p_score, batch_pad.reshape(1, NP))
    maxp = _sc_max(xw.reshape(NP * D), batch_pad.reshape(NTILES, RPT))
    out = _call_head(sums, cnts, maxp.reshape(NTILES, G, D),
                     Wc1, bc1, Wc2, bc2)
    return out


# speedup vs baseline: 1.3543x; 1.0056x over previous
import functools

import jax
import jax.numpy as jnp
from jax import lax
from jax.experimental import pallas as pl
from jax.experimental.pallas import tpu as pltpu, tpu_sc as plsc

N = 10000
NP = 10240
E = 320000
D = 128
G = 64
KSEL = N // 2
NCORES = 2
NSUB = 16
NTILES = NCORES * NSUB
EPT = E // NTILES
ECH = 128
NCH = (EPT + ECH - 1) // ECH
EPAD = NCH * ECH
DH = D // 2
EPT2 = E // NSUB
NCH2 = 158
EPAD2 = NCH2 * ECH
RPT = NP // NTILES
INT_MIN = -(2 ** 31)

_mesh = plsc.VectorSubcoreMesh(core_axis_name="c", subcore_axis_name="s")


@functools.partial(
    pl.kernel,
    out_type=jax.ShapeDtypeStruct((NTILES, NP), jnp.float32),
    mesh=_mesh,
    scratch_types=[
        pltpu.VMEM((EPAD,), jnp.int32),
        pltpu.VMEM((NP,), jnp.float32),
    ],
    compiler_params=pltpu.CompilerParams(needs_layout_passes=False),
)
def _sc_deg(dst_hbm, out_hbm, dstv, deg_l):
    c = lax.axis_index("c")
    s = lax.axis_index("s")
    wid = c * NSUB + s
    pltpu.sync_copy(dst_hbm.at[wid], dstv)
    def _zero(i, _):
        deg_l[pl.ds(i * 16, 16)] = jnp.zeros((16,), jnp.float32)
        return 0
    lax.fori_loop(0, NP // 16, _zero, 0)
    ones = jnp.ones((16,), jnp.float32)
    def _grp(g, _):
        ii = dstv[pl.ds(g * 16, 16)]
        plsc.addupdate_scatter(deg_l, [ii], ones)
        return 0
    lax.fori_loop(0, EPAD // 16, _grp, 0)
    pltpu.sync_copy(deg_l, out_hbm.at[wid])


@functools.partial(
    pl.kernel,
    out_type=jax.ShapeDtypeStruct((NCORES, NP, DH), jnp.float32),
    mesh=_mesh,
    scratch_types=[
        pltpu.VMEM((NCH2, ECH), jnp.int32),
        pltpu.VMEM((NCH2, ECH), jnp.int32),
        pltpu.VMEM((ECH, DH), jnp.float32),
        pltpu.VMEM((ECH, DH), jnp.float32),
        pltpu.VMEM_SHARED((NP, DH), jnp.float32),
        pltpu.SemaphoreType.DMA,
        pltpu.SemaphoreType.DMA,
    ],
    compiler_params=pltpu.CompilerParams(use_tc_tiling_on_sc=False),
)
def _sc_agg(z_hbm, src_hbm, dst_hbm, zero_hbm, out_hbm,
            srcv, dstv, rows0, rows1, agg_sh, sem0, sem1):
    c = lax.axis_index("c")
    s = lax.axis_index("s")
    zc = z_hbm.at[c]
    pltpu.sync_copy(src_hbm.at[s], srcv)
    pltpu.sync_copy(dst_hbm.at[s], dstv)
    pltpu.sync_copy(zero_hbm, agg_sh.at[pl.ds(s * (NP // NSUB), NP // NSUB)])
    plsc.subcore_barrier()
    pltpu.async_copy(zc.at[srcv.at[0]], rows0, sem0)
    def _pair(k, _):
        i0 = 2 * k
        pltpu.async_copy(zc.at[srcv.at[i0 + 1]], rows1, sem1)
        pltpu.make_async_copy(zc.at[srcv.at[i0]], rows0, sem0).wait()
        pltpu.sync_copy(rows0, agg_sh.at[dstv.at[i0]], add=True)
        pltpu.async_copy(zc.at[srcv.at[i0 + 2]], rows0, sem0)
        pltpu.make_async_copy(zc.at[srcv.at[i0 + 1]], rows1, sem1).wait()
        pltpu.sync_copy(rows1, agg_sh.at[dstv.at[i0 + 1]], add=True)
        return 0
    lax.fori_loop(0, (NCH2 - 2) // 2, _pair, 0)
    pltpu.make_async_copy(zc.at[srcv.at[NCH2 - 2]], rows0, sem0).wait()
    pltpu.sync_copy(rows0, agg_sh.at[dstv.at[NCH2 - 2]], add=True)
    pltpu.async_copy(zc.at[srcv.at[NCH2 - 1]], rows1, sem1)
    pltpu.make_async_copy(zc.at[srcv.at[NCH2 - 1]], rows1, sem1).wait()
    pltpu.sync_copy(rows1, agg_sh.at[dstv.at[NCH2 - 1]], add=True)
    plsc.subcore_barrier()
    pltpu.sync_copy(
        agg_sh.at[pl.ds(s * (NP // NSUB), NP // NSUB)],
        out_hbm.at[c, pl.ds(s * (NP // NSUB), NP // NSUB)],
    )


@functools.partial(
    pl.kernel,
    out_type=jax.ShapeDtypeStruct((NTILES, G * D), jnp.float32),
    mesh=_mesh,
    scratch_types=[
        pltpu.VMEM((RPT * D,), jnp.float32),
        pltpu.VMEM((RPT,), jnp.int32),
        pltpu.VMEM((G * D,), jnp.float32),
    ],
    compiler_params=pltpu.CompilerParams(needs_layout_passes=False),
)
def _sc_max(xw_hbm, batch_hbm, out_hbm, rowsf, batchv, acc_m):
    c = lax.axis_index("c")
    s = lax.axis_index("s")
    wid = c * NSUB + s
    pltpu.sync_copy(xw_hbm.at[pl.ds(wid * RPT * D, RPT * D)], rowsf)
    pltpu.sync_copy(batch_hbm.at[wid], batchv)
    def _zero(i, _):
        acc_m[pl.ds(i * 16, 16)] = jnp.zeros((16,), jnp.float32)
        return 0
    lax.fori_loop(0, G * D // 16, _zero, 0)
    def _group(g, _):
        bb = batchv[pl.ds(g * 16, 16)]
        for r in range(16):
            base = bb[r] * D
            i = g * 16 + r
            for j in range(D // 16):
                ch = rowsf[pl.ds(i * D + j * 16, 16)]
                idx = base + j * 16 + lax.iota(jnp.int32, 16)
                cur = plsc.load_gather(acc_m, [idx])
                plsc.store_scatter(acc_m, [idx], jnp.maximum(cur, ch))
        return 0
    lax.fori_loop(0, RPT // 16, _group, 0)
    pltpu.sync_copy(acc_m, out_hbm.at[wid])


def _tc_pre(dt_ref, x_ref, rdeg_ref, z_ref):
    deg = jnp.sum(dt_ref[...], axis=1, keepdims=True)
    rdeg = lax.rsqrt(jnp.maximum(deg, 1.0))
    rdeg_ref[...] = rdeg
    z = x_ref[...] * rdeg
    z_ref[0] = z[:, :DH]
    z_ref[1] = z[:, DH:]


def _tc_conv(sp_ref, rdeg_ref, w_ref, b_ref, o_ref, *, scale_out):
    a = jnp.concatenate([sp_ref[0], sp_ref[1]], axis=1) * rdeg_ref[...]
    h = jnp.dot(a, w_ref[...], preferred_element_type=jnp.float32)
    f = jnp.maximum(h + b_ref[...], 0.0)
    if scale_out:
        f = f * rdeg_ref[...]
        o_ref[0] = f[:, :DH]
        o_ref[1] = f[:, DH:]
    else:
        o_ref[...] = f


def _tc_thresh(sp_ref, rdeg_ref, w_ref, b_ref, p_ref, batch_ref,
               xw_ref, sums_ref, cnts_ref):
    a = jnp.concatenate([sp_ref[0], sp_ref[1]], axis=1) * rdeg_ref[...]
    h = jnp.dot(a, w_ref[...], preferred_element_type=jnp.float32)
    f = jnp.maximum(h + b_ref[...], 0.0)
    p = p_ref[...]
    pn = p * lax.rsqrt(jnp.sum(p * p))
    score = jnp.sum(f * pn, axis=1, keepdims=True)
    u = lax.bitcast_convert_type(score, jnp.int32)
    key = jnp.where(u >= 0, u, u ^ jnp.int32(0x7FFFFFFF))
    ridx = lax.broadcasted_iota(jnp.int32, (NP, 1), 0)
    valid = ridx < N
    key = jnp.where(valid, key, jnp.int32(INT_MIN))

    def _bis(_, carry):
        lo, hi = carry
        mid = (lo & hi) + ((lo ^ hi) >> 1)
        cnt = jnp.sum((key >= mid).astype(jnp.int32))
        ge = cnt >= KSEL
        return (jnp.where(ge, mid, lo), jnp.where(ge, hi, mid))

    lo, _ = lax.fori_loop(0, 32, _bis, (jnp.int32(INT_MIN),
                                        jnp.int32(0x7F800000)))
    sel = jnp.logical_and(valid, key >= lo)
    w = jnp.where(sel, jnp.maximum(score, 0.0), 0.0)
    xw = f * w
    xw_ref[...] = xw
    gI = lax.broadcasted_iota(jnp.int32, (G, NP), 0)
    oh = (gI == batch_ref[...]).astype(jnp.float32)
    sums_ref[...] = jnp.dot(oh, xw, preferred_element_type=jnp.float32)
    cnts_ref[...] = jnp.dot(oh, sel.astype(jnp.float32),
                            preferred_element_type=jnp.float32)


def _tc_head(sums_ref, cnts_ref, maxp_ref, w1_ref, b1_ref, w2_ref, b2_ref,
             o_ref):
    m = maxp_ref[0]
    for i in range(1, NTILES):
        m = jnp.maximum(m, maxp_ref[i])
    m = jnp.maximum(m, 0.0)
    cnt = jnp.maximum(cnts_ref[...], 1.0)
    mean = jnp.maximum(sums_ref[...] / cnt, 0.0)
    readout = jnp.concatenate([mean, m], axis=1)
    h = jnp.dot(readout, w1_ref[...], preferred_element_type=jnp.float32)
    h = jnp.maximum(h + b1_ref[...], 0.0)
    o_ref[...] = jnp.dot(h, w2_ref[...],
                         preferred_element_type=jnp.float32) + b2_ref[...]


def _call_pre(degt, x_pad):
    return pl.pallas_call(
        _tc_pre,
        out_shape=(jax.ShapeDtypeStruct((NP, 1), jnp.float32),
                   jax.ShapeDtypeStruct((NCORES, NP, DH), jnp.float32)),
    )(degt, x_pad)


def _call_conv(sp, rdeg, w, b, scale_out):
    out_shape = (jax.ShapeDtypeStruct((NCORES, NP, DH), jnp.float32)
                 if scale_out else jax.ShapeDtypeStruct((NP, D), jnp.float32))
    return pl.pallas_call(
        functools.partial(_tc_conv, scale_out=scale_out),
        out_shape=out_shape,
    )(sp, rdeg, w, b.reshape(1, D))


def _call_thresh(sp, rdeg, w, b, p, batch_row):
    return pl.pallas_call(
        _tc_thresh,
        out_shape=(jax.ShapeDtypeStruct((NP, D), jnp.float32),
                   jax.ShapeDtypeStruct((G, D), jnp.float32),
                   jax.ShapeDtypeStruct((G, 1), jnp.float32)),
    )(sp, rdeg, w, b.reshape(1, D), p.reshape(1, D), batch_row)


def _call_head(sums, cnts, maxp, w1, b1, w2, b2):
    return pl.pallas_call(
        _tc_head,
        out_shape=jax.ShapeDtypeStruct((G, 10), jnp.float32),
    )(sums, cnts, maxp, w1, b1.reshape(1, D), w2, b2.reshape(1, 10))


def kernel(x, edge_index, batch, Wt0, bt0, Wt1, bt1, Wg00, bg00, Wg01, bg01,
           Wg10, bg10, Wg11, bg11, p_score, Wc1, bc1, Wc2, bc2):
    src = edge_index[0].reshape(NSUB, EPT2)
    dst = edge_index[1].reshape(NSUB, EPT2)
    pad_w = EPAD2 - EPT2
    pad_dst = (N + (jnp.arange(NSUB * pad_w, dtype=jnp.int32) % (NP - N))
               ).reshape(NSUB, pad_w)
    pad_src = (jnp.arange(NSUB * pad_w, dtype=jnp.int32) % N
               ).reshape(NSUB, pad_w)
    src_slab = jnp.concatenate([src, pad_src], axis=1).reshape(
        NSUB, NCH2, ECH)
    dst_slab = jnp.concatenate([dst, pad_dst], axis=1).reshape(
        NSUB, NCH2, ECH)
    x_pad = jnp.pad(x, ((0, NP - N), (0, 0)))
    zero_d = jnp.zeros((NP // NSUB, DH), jnp.float32)

    dst_deg = jnp.pad(edge_index[1].reshape(NTILES, EPT),
                      ((0, 0), (0, EPAD - EPT)), constant_values=N)
    degp = _sc_deg(dst_deg)
    rdeg, z = _call_pre(degp.T, x_pad)

    convs = ((Wt0, bt0), (Wt1, bt1), (Wg00, bg00))
    for w, b in convs:
        sp = _sc_agg(z, src_slab, dst_slab, zero_d)
        z = _call_conv(sp, rdeg, w, b, True)
    sp = _sc_agg(z, src_slab, dst_slab, zero_d)

    batch_pad = jnp.pad(batch, (0, NP - N), constant_values=G - 1)
    xw, sums, cnts = _call_thresh(sp, rdeg, Wg01, bg01, p_score,
                                  batch_pad.reshape(1, NP))
    maxp = _sc_max(xw.reshape(NP * D), batch_pad.reshape(NTILES, RPT))
    out = _call_head(sums, cnts, maxp.reshape(NTILES, G, D),
                     Wc1, bc1, Wc2, bc2)
    return out


# speedup vs baseline: 1.3752x; 1.0154x over previous
import functools

import jax
import jax.numpy as jnp
from jax import lax
from jax.experimental import pallas as pl
from jax.experimental.pallas import tpu as pltpu, tpu_sc as plsc

N = 10000
NP = 10240
E = 320000
D = 128
G = 64
KSEL = N // 2
NCORES = 2
NSUB = 16
NTILES = NCORES * NSUB
EPT = E // NTILES
ECH = 128
NCH = (EPT + ECH - 1) // ECH
EPAD = NCH * ECH
DH = D // 2
EPT2 = E // NSUB
NCH2 = 158
EPAD2 = NCH2 * ECH
RPT = NP // NTILES
INT_MIN = -(2 ** 31)

_mesh = plsc.VectorSubcoreMesh(core_axis_name="c", subcore_axis_name="s")


@functools.partial(
    pl.kernel,
    out_type=jax.ShapeDtypeStruct((NTILES, NP), jnp.float32),
    mesh=_mesh,
    scratch_types=[
        pltpu.VMEM((EPAD,), jnp.int32),
        pltpu.VMEM((NP,), jnp.float32),
    ],
    compiler_params=pltpu.CompilerParams(needs_layout_passes=False),
)
def _sc_deg(dst_hbm, out_hbm, dstv, deg_l):
    c = lax.axis_index("c")
    s = lax.axis_index("s")
    wid = c * NSUB + s
    pltpu.sync_copy(dst_hbm.at[wid], dstv)
    def _zero(i, _):
        deg_l[pl.ds(i * 16, 16)] = jnp.zeros((16,), jnp.float32)
        return 0
    lax.fori_loop(0, NP // 16, _zero, 0)
    ones = jnp.ones((16,), jnp.float32)
    def _grp(g, _):
        ii = dstv[pl.ds(g * 16, 16)]
        plsc.addupdate_scatter(deg_l, [ii], ones)
        return 0
    lax.fori_loop(0, EPAD // 16, _grp, 0)
    pltpu.sync_copy(deg_l, out_hbm.at[wid])


@functools.partial(
    pl.kernel,
    out_type=jax.ShapeDtypeStruct((NCORES, NP, DH), jnp.float32),
    mesh=_mesh,
    scratch_types=[
        pltpu.VMEM((NCH2, ECH), jnp.int32),
        pltpu.VMEM((NCH2, ECH), jnp.int32),
        pltpu.VMEM((ECH, DH), jnp.float32),
        pltpu.VMEM((ECH, DH), jnp.float32),
        pltpu.VMEM_SHARED((NP, DH), jnp.float32),
        pltpu.SemaphoreType.DMA,
        pltpu.SemaphoreType.DMA,
        pltpu.SemaphoreType.DMA,
    ],
    compiler_params=pltpu.CompilerParams(use_tc_tiling_on_sc=False),
)
def _sc_agg(z_hbm, src_hbm, dst_hbm, zero_hbm, out_hbm,
            srcv, dstv, rows0, rows1, agg_sh, sem0, sem1, sem2):
    c = lax.axis_index("c")
    s = lax.axis_index("s")
    zc = z_hbm.at[c]
    a = pltpu.async_copy(src_hbm.at[s], srcv, sem0)
    b = pltpu.async_copy(dst_hbm.at[s], dstv, sem1)
    zi = pltpu.async_copy(
        zero_hbm, agg_sh.at[pl.ds(s * (NP // NSUB), NP // NSUB)], sem2)
    a.wait()
    pltpu.async_copy(zc.at[srcv.at[0]], rows0, sem0)
    b.wait()
    zi.wait()
    plsc.subcore_barrier()
    def _pair(k, _):
        i0 = 2 * k
        pltpu.async_copy(zc.at[srcv.at[i0 + 1]], rows1, sem1)
        pltpu.make_async_copy(zc.at[srcv.at[i0]], rows0, sem0).wait()
        pltpu.sync_copy(rows0, agg_sh.at[dstv.at[i0]], add=True)
        pltpu.async_copy(zc.at[srcv.at[i0 + 2]], rows0, sem0)
        pltpu.make_async_copy(zc.at[srcv.at[i0 + 1]], rows1, sem1).wait()
        pltpu.sync_copy(rows1, agg_sh.at[dstv.at[i0 + 1]], add=True)
        return 0
    lax.fori_loop(0, (NCH2 - 2) // 2, _pair, 0)
    pltpu.make_async_copy(zc.at[srcv.at[NCH2 - 2]], rows0, sem0).wait()
    pltpu.sync_copy(rows0, agg_sh.at[dstv.at[NCH2 - 2]], add=True)
    pltpu.async_copy(zc.at[srcv.at[NCH2 - 1]], rows1, sem1)
    pltpu.make_async_copy(zc.at[srcv.at[NCH2 - 1]], rows1, sem1).wait()
    pltpu.sync_copy(rows1, agg_sh.at[dstv.at[NCH2 - 1]], add=True)
    plsc.subcore_barrier()
    pltpu.sync_copy(
        agg_sh.at[pl.ds(s * (NP // NSUB), NP // NSUB)],
        out_hbm.at[c, pl.ds(s * (NP // NSUB), NP // NSUB)],
    )


@functools.partial(
    pl.kernel,
    out_type=jax.ShapeDtypeStruct((NTILES, G * D), jnp.float32),
    mesh=_mesh,
    scratch_types=[
        pltpu.VMEM((RPT * D,), jnp.float32),
        pltpu.VMEM((RPT,), jnp.int32),
        pltpu.VMEM((G * D,), jnp.float32),
    ],
    compiler_params=pltpu.CompilerParams(needs_layout_passes=False),
)
def _sc_max(xw_hbm, batch_hbm, out_hbm, rowsf, batchv, acc_m):
    c = lax.axis_index("c")
    s = lax.axis_index("s")
    wid = c * NSUB + s
    pltpu.sync_copy(xw_hbm.at[pl.ds(wid * RPT * D, RPT * D)], rowsf)
    pltpu.sync_copy(batch_hbm.at[wid], batchv)
    def _zero(i, _):
        acc_m[pl.ds(i * 16, 16)] = jnp.zeros((16,), jnp.float32)
        return 0
    lax.fori_loop(0, G * D // 16, _zero, 0)
    def _group(g, _):
        bb = batchv[pl.ds(g * 16, 16)]
        for r in range(16):
            base = bb[r] * D
            i = g * 16 + r
            for j in range(D // 16):
                ch = rowsf[pl.ds(i * D + j * 16, 16)]
                idx = base + j * 16 + lax.iota(jnp.int32, 16)
                cur = plsc.load_gather(acc_m, [idx])
                plsc.store_scatter(acc_m, [idx], jnp.maximum(cur, ch))
        return 0
    lax.fori_loop(0, RPT // 16, _group, 0)
    pltpu.sync_copy(acc_m, out_hbm.at[wid])


def _tc_pre(dt_ref, x_ref, rdeg_ref, z_ref):
    deg = jnp.sum(dt_ref[...], axis=1, keepdims=True)
    rdeg = lax.rsqrt(jnp.maximum(deg, 1.0))
    rdeg_ref[...] = rdeg
    z = x_ref[...] * rdeg
    z_ref[0] = z[:, :DH]
    z_ref[1] = z[:, DH:]


def _tc_conv(sp_ref, rdeg_ref, w_ref, b_ref, o_ref, *, scale_out):
    a = jnp.concatenate([sp_ref[0], sp_ref[1]], axis=1) * rdeg_ref[...]
    h = jnp.dot(a, w_ref[...], preferred_element_type=jnp.float32)
    f = jnp.maximum(h + b_ref[...], 0.0)
    if scale_out:
        f = f * rdeg_ref[...]
        o_ref[0] = f[:, :DH]
        o_ref[1] = f[:, DH:]
    else:
        o_ref[...] = f


def _tc_thresh(sp_ref, rdeg_ref, w_ref, b_ref, p_ref, batch_ref,
               xw_ref, sums_ref, cnts_ref):
    a = jnp.concatenate([sp_ref[0], sp_ref[1]], axis=1) * rdeg_ref[...]
    h = jnp.dot(a, w_ref[...], preferred_element_type=jnp.float32)
    f = jnp.maximum(h + b_ref[...], 0.0)
    p = p_ref[...]
    pn = p * lax.rsqrt(jnp.sum(p * p))
    score = jnp.sum(f * pn, axis=1, keepdims=True)
    u = lax.bitcast_convert_type(score, jnp.int32)
    key = jnp.where(u >= 0, u, u ^ jnp.int32(0x7FFFFFFF))
    ridx = lax.broadcasted_iota(jnp.int32, (NP, 1), 0)
    valid = ridx < N
    key = jnp.where(valid, key, jnp.int32(INT_MIN))

    def _bis(_, carry):
        lo, hi = carry
        mid = (lo & hi) + ((lo ^ hi) >> 1)
        cnt = jnp.sum((key >= mid).astype(jnp.int32))
        ge = cnt >= KSEL
        return (jnp.where(ge, mid, lo), jnp.where(ge, hi, mid))

    lo, _ = lax.fori_loop(0, 32, _bis, (jnp.int32(INT_MIN),
                                        jnp.int32(0x7F800000)))
    sel = jnp.logical_and(valid, key >= lo)
    w = jnp.where(sel, jnp.maximum(score, 0.0), 0.0)
    xw = f * w
    xw_ref[...] = xw
    gI = lax.broadcasted_iota(jnp.int32, (G, NP), 0)
    oh = (gI == batch_ref[...]).astype(jnp.float32)
    sums_ref[...] = jnp.dot(oh, xw, preferred_element_type=jnp.float32)
    cnts_ref[...] = jnp.dot(oh, sel.astype(jnp.float32),
                            preferred_element_type=jnp.float32)


def _tc_head(sums_ref, cnts_ref, maxp_ref, w1_ref, b1_ref, w2_ref, b2_ref,
             o_ref):
    m = maxp_ref[0]
    for i in range(1, NTILES):
        m = jnp.maximum(m, maxp_ref[i])
    m = jnp.maximum(m, 0.0)
    cnt = jnp.maximum(cnts_ref[...], 1.0)
    mean = jnp.maximum(sums_ref[...] / cnt, 0.0)
    readout = jnp.concatenate([mean, m], axis=1)
    h = jnp.dot(readout, w1_ref[...], preferred_element_type=jnp.float32)
    h = jnp.maximum(h + b1_ref[...], 0.0)
    o_ref[...] = jnp.dot(h, w2_ref[...],
                         preferred_element_type=jnp.float32) + b2_ref[...]


def _call_pre(degt, x_pad):
    return pl.pallas_call(
        _tc_pre,
        out_shape=(jax.ShapeDtypeStruct((NP, 1), jnp.float32),
                   jax.ShapeDtypeStruct((NCORES, NP, DH), jnp.float32)),
    )(degt, x_pad)


def _call_conv(sp, rdeg, w, b, scale_out):
    out_shape = (jax.ShapeDtypeStruct((NCORES, NP, DH), jnp.float32)
                 if scale_out else jax.ShapeDtypeStruct((NP, D), jnp.float32))
    return pl.pallas_call(
        functools.partial(_tc_conv, scale_out=scale_out),
        out_shape=out_shape,
    )(sp, rdeg, w, b.reshape(1, D))


def _call_thresh(sp, rdeg, w, b, p, batch_row):
    return pl.pallas_call(
        _tc_thresh,
        out_shape=(jax.ShapeDtypeStruct((NP, D), jnp.float32),
                   jax.ShapeDtypeStruct((G, D), jnp.float32),
                   jax.ShapeDtypeStruct((G, 1), jnp.float32)),
    )(sp, rdeg, w, b.reshape(1, D), p.reshape(1, D), batch_row)


def _call_head(sums, cnts, maxp, w1, b1, w2, b2):
    return pl.pallas_call(
        _tc_head,
        out_shape=jax.ShapeDtypeStruct((G, 10), jnp.float32),
    )(sums, cnts, maxp, w1, b1.reshape(1, D), w2, b2.reshape(1, 10))


def kernel(x, edge_index, batch, Wt0, bt0, Wt1, bt1, Wg00, bg00, Wg01, bg01,
           Wg10, bg10, Wg11, bg11, p_score, Wc1, bc1, Wc2, bc2):
    src = edge_index[0].reshape(NSUB, EPT2)
    dst = edge_index[1].reshape(NSUB, EPT2)
    pad_w = EPAD2 - EPT2
    pad_dst = (N + (jnp.arange(NSUB * pad_w, dtype=jnp.int32) % (NP - N))
               ).reshape(NSUB, pad_w)
    pad_src = (jnp.arange(NSUB * pad_w, dtype=jnp.int32) % N
               ).reshape(NSUB, pad_w)
    src_slab = jnp.concatenate([src, pad_src], axis=1).reshape(
        NSUB, NCH2, ECH)
    dst_slab = jnp.concatenate([dst, pad_dst], axis=1).reshape(
        NSUB, NCH2, ECH)
    x_pad = jnp.pad(x, ((0, NP - N), (0, 0)))
    zero_d = jnp.zeros((NP // NSUB, DH), jnp.float32)

    dst_deg = jnp.pad(edge_index[1].reshape(NTILES, EPT),
                      ((0, 0), (0, EPAD - EPT)), constant_values=N)
    degp = _sc_deg(dst_deg)
    rdeg, z = _call_pre(degp.T, x_pad)

    convs = ((Wt0, bt0), (Wt1, bt1), (Wg00, bg00))
    for w, b in convs:
        sp = _sc_agg(z, src_slab, dst_slab, zero_d)
        z = _call_conv(sp, rdeg, w, b, True)
    sp = _sc_agg(z, src_slab, dst_slab, zero_d)

    batch_pad = jnp.pad(batch, (0, NP - N), constant_values=G - 1)
    xw, sums, cnts = _call_thresh(sp, rdeg, Wg01, bg01, p_score,
                                  batch_pad.reshape(1, NP))
    maxp = _sc_max(xw.reshape(NP * D), batch_pad.reshape(NTILES, RPT))
    out = _call_head(sums, cnts, maxp.reshape(NTILES, G, D),
                     Wc1, bc1, Wc2, bc2)
    return out
